# Initial kernel scaffold; baseline (speedup 1.0000x reference)
import jax
import jax.numpy as jnp
from jax.experimental import pallas as pl


def kernel(z, emb_weight):
    raise NotImplementedError("write your pallas kernel here")


# speedup vs baseline: 6.1719x; 6.1719x over previous
import functools

import jax
import jax.numpy as jnp
from jax import lax
from jax.experimental import pallas as pl
from jax.experimental.pallas import tpu as pltpu
from jax.experimental.pallas import tpu_sc as plsc

V = 8192
D = 32
B = 16 * 1024

_NC, _NS = 2, 16
_NW = _NC * _NS
_BPW = B // _NW
_CHUNK = 128
_NCH = _BPW // _CHUNK


@functools.cache
def _make_gather_rows():
    @functools.partial(
        pl.kernel,
        out_type=jax.ShapeDtypeStruct((B, D), jnp.float32),
        mesh=plsc.VectorSubcoreMesh(core_axis_name="c", subcore_axis_name="s",
                                    num_cores=_NC, num_subcores=_NS),
        scratch_types=[
            pltpu.VMEM((_NCH, _CHUNK), jnp.int32),
            pltpu.VMEM((_BPW, D), jnp.float32),
            pltpu.SemaphoreType.DMA,
        ],
        compiler_params=pltpu.CompilerParams(use_tc_tiling_on_sc=False),
    )
    def _gather_rows(idx_hbm, table_hbm, out_hbm, idx_v, rows_v, sem):
        wid = lax.axis_index("s") * _NC + lax.axis_index("c")
        base = wid * _BPW
        pltpu.sync_copy(idx_hbm.at[wid], idx_v)
        copies = [
            pltpu.async_copy(table_hbm.at[idx_v.at[k]],
                             rows_v.at[pl.ds(k * _CHUNK, _CHUNK)], sem)
            for k in range(_NCH)
        ]
        for cp in copies:
            cp.wait()
        pltpu.sync_copy(rows_v, out_hbm.at[pl.ds(base, _BPW)])

    return _gather_rows


def _loss_body(q_ref, z_ref, out_ref):
    i = pl.program_id(0)
    dq = q_ref[0] - z_ref[0]
    part = jnp.sum(dq * dq)

    @pl.when(i == 0)
    def _():
        out_ref[0, 0] = part

    @pl.when(i > 0)
    def _():
        out_ref[0, 0] = out_ref[0, 0] + part


_loss_call = pl.pallas_call(
    _loss_body,
    grid=(16,),
    in_specs=[
        pl.BlockSpec((1, 1024, D), lambda i: (i, 0, 0)),
        pl.BlockSpec((1, 1024, D), lambda i: (i, 0, 0)),
    ],
    out_specs=pl.BlockSpec((1, 1), lambda i: (0, 0), memory_space=pltpu.SMEM),
    out_shape=jax.ShapeDtypeStruct((1, 1), jnp.float32),
    compiler_params=pltpu.CompilerParams(dimension_semantics=("arbitrary",)),
)


def kernel(z, emb_weight):
    flat_z = z.reshape(-1, emb_weight.shape[1])
    mm = lax.dot_general(flat_z.astype(jnp.bfloat16), emb_weight,
                         (((1,), (1,)), ((), ())),
                         preferred_element_type=jnp.float32)
    distance = (jnp.sum(flat_z ** 2, axis=1, keepdims=True)
                + jnp.sum(emb_weight ** 2, axis=1)
                - 2.0 * mm)
    idx = jnp.argmin(distance, axis=1)
    onehot = (lax.broadcasted_iota(jnp.int32, (B, V), 1)
              == idx[:, None]).astype(jnp.float32)
    s = jnp.sum(onehot)
    q = _make_gather_rows()(idx.reshape(_NW, _NCH, _CHUNK), emb_weight)
    q = q.reshape(z.shape)
    loss_sum = _loss_call(q, z)
    loss = 0.5 * loss_sum[0, 0] / z.size + (s - s)
    quantized_st = z + (q - z)
    return quantized_st, loss, loss


# speedup vs baseline: 8.8292x; 1.4306x over previous
import functools

import jax
import jax.numpy as jnp
from jax import lax
from jax.experimental import pallas as pl
from jax.experimental.pallas import tpu as pltpu
from jax.experimental.pallas import tpu_sc as plsc

V = 8192
D = 32
B = 16 * 1024

_NC, _NS = 2, 16
_NW = _NC * _NS
_BPW = B // _NW
_CHUNK = 128
_NCH = _BPW // _CHUNK


@functools.cache
def _make_gather_rows():
    @functools.partial(
        pl.kernel,
        out_type=jax.ShapeDtypeStruct((B, D), jnp.float32),
        mesh=plsc.VectorSubcoreMesh(core_axis_name="c", subcore_axis_name="s",
                                    num_cores=_NC, num_subcores=_NS),
        scratch_types=[
            pltpu.VMEM((_NCH, _CHUNK), jnp.int32),
            pltpu.VMEM((_BPW, D), jnp.float32),
            pltpu.SemaphoreType.DMA,
        ],
        compiler_params=pltpu.CompilerParams(use_tc_tiling_on_sc=False),
    )
    def _gather_rows(idx_hbm, table_hbm, out_hbm, idx_v, rows_v, sem):
        wid = lax.axis_index("s") * _NC + lax.axis_index("c")
        base = wid * _BPW
        pltpu.sync_copy(idx_hbm.at[wid], idx_v)
        copies = [
            pltpu.async_copy(table_hbm.at[idx_v.at[k]],
                             rows_v.at[pl.ds(k * _CHUNK, _CHUNK)], sem)
            for k in range(_NCH)
        ]
        for cp in copies:
            cp.wait()
        pltpu.sync_copy(rows_v, out_hbm.at[pl.ds(base, _BPW)])

    return _gather_rows


def _loss_body(q_ref, z_ref, out_ref):
    i = pl.program_id(0)
    dq = q_ref[0] - z_ref[0]
    part = jnp.sum(dq * dq)

    @pl.when(i == 0)
    def _():
        out_ref[0, 0] = part

    @pl.when(i > 0)
    def _():
        out_ref[0, 0] = out_ref[0, 0] + part


_loss_call = pl.pallas_call(
    _loss_body,
    grid=(16,),
    in_specs=[
        pl.BlockSpec((1, 1024, D), lambda i: (i, 0, 0)),
        pl.BlockSpec((1, 1024, D), lambda i: (i, 0, 0)),
    ],
    out_specs=pl.BlockSpec((1, 1), lambda i: (0, 0), memory_space=pltpu.SMEM),
    out_shape=jax.ShapeDtypeStruct((1, 1), jnp.float32),
    compiler_params=pltpu.CompilerParams(dimension_semantics=("arbitrary",)),
)


def kernel(z, emb_weight):
    flat_z = z.reshape(-1, emb_weight.shape[1])
    mm = lax.dot_general(flat_z.astype(jnp.bfloat16), emb_weight,
                         (((1,), (1,)), ((), ())),
                         preferred_element_type=jnp.float32)
    distance = (jnp.sum(flat_z ** 2, axis=1, keepdims=True)
                + jnp.sum(emb_weight ** 2, axis=1)
                - 2.0 * mm)
    idx = jnp.argmin(distance, axis=1)
    q = _make_gather_rows()(idx.reshape(_NW, _NCH, _CHUNK), emb_weight)
    q = q.reshape(z.shape)
    loss_sum = _loss_call(q, z)
    loss = 0.5 * loss_sum[0, 0] / z.size
    quantized_st = z + (q - z)
    return quantized_st, loss, loss
